# Initial kernel scaffold; baseline (speedup 1.0000x reference)
import jax
import jax.numpy as jnp
from jax.experimental import pallas as pl


def kernel(x, p):
    raise NotImplementedError("write your pallas kernel here")


# speedup vs baseline: 4.0842x; 4.0842x over previous
import jax
import jax.numpy as jnp
from jax.experimental import pallas as pl
from jax.experimental.pallas import tpu as pltpu

_EPS = 1e-6
_CHUNK = 64


def _mom_kernel(x_ref, p_ref, o_ref):
    T = x_ref.shape[1]
    C = x_ref.shape[2]
    nck = T // _CHUNK
    g = _CHUNK // 8
    a1 = jnp.zeros((8, C), jnp.float32)
    a2 = jnp.zeros((8, C), jnp.float32)
    a3 = jnp.zeros((8, C), jnp.float32)
    a4 = jnp.zeros((8, C), jnp.float32)
    for k in range(nck):
        xc = x_ref[0, k * _CHUNK:(k + 1) * _CHUNK, :]
        x2 = xc * xc
        x3 = x2 * xc
        x4 = x2 * x2
        a1 = a1 + jnp.sum(xc.reshape(g, 8, C), axis=0)
        a2 = a2 + jnp.sum(x2.reshape(g, 8, C), axis=0)
        a3 = a3 + jnp.sum(x3.reshape(g, 8, C), axis=0)
        a4 = a4 + jnp.sum(x4.reshape(g, 8, C), axis=0)
    inv_n = 1.0 / T
    m1 = jnp.sum(a1, axis=0, keepdims=True) * inv_n
    m2 = jnp.sum(a2, axis=0, keepdims=True) * inv_n
    m3 = jnp.sum(a3, axis=0, keepdims=True) * inv_n
    m4 = jnp.sum(a4, axis=0, keepdims=True) * inv_n
    var = m2 - m1 * m1
    m1sq = m1 * m1
    cm3 = m3 - 3.0 * m1 * m2 + 2.0 * m1 * m1sq
    cm4 = m4 - 4.0 * m1 * m3 + 6.0 * m1sq * m2 - 3.0 * m1sq * m1sq
    v_eps = var + _EPS
    std = jnp.sqrt(v_eps)
    skew = cm3 / (v_eps * std)
    kurt = cm4 / (v_eps * v_eps)
    o_ref[0, 1:2, :] = var
    o_ref[0, 2:3, :] = skew
    o_ref[0, 3:4, :] = kurt

    pv = p_ref[0]

    @pl.when(pv == 1.0)
    def _():
        o_ref[0, 0:1, :] = m1

    @pl.when(pv != 1.0)
    def _():
        ap = jnp.zeros((8, C), jnp.float32)
        for k in range(nck):
            xc = x_ref[0, k * _CHUNK:(k + 1) * _CHUNK, :]
            xp = jnp.exp2(pv * jnp.log2(xc))
            ap = ap + jnp.sum(xp.reshape(g, 8, C), axis=0)
        o_ref[0, 0:1, :] = jnp.sum(ap, axis=0, keepdims=True) * inv_n


def _moments(x, p, *, interpret=False):
    B, T, C = x.shape
    out3 = pl.pallas_call(
        _mom_kernel,
        out_shape=jax.ShapeDtypeStruct((B, 4, C), jnp.float32),
        grid=(B,),
        in_specs=[
            pl.BlockSpec((1, T, C), lambda b: (b, 0, 0)),
            pl.BlockSpec(memory_space=pltpu.SMEM),
        ],
        out_specs=pl.BlockSpec((1, 4, C), lambda b: (b, 0, 0)),
        compiler_params=pltpu.CompilerParams(
            dimension_semantics=("parallel",),
        ),
        name="mixed_hom_moments",
        interpret=interpret,
    )(x, p)
    return out3.reshape(B, 4 * C)


def kernel(x, p):
    return _moments(x.astype(jnp.float32), p.astype(jnp.float32))
